# baseline (speedup 1.0000x reference)
import functools

import jax
import jax.numpy as jnp
from jax.experimental import pallas as pl
from jax.experimental.pallas import tpu as pltpu

N = 4096
NFEAT = 4096
NCLASS = 8
BLK = 256


def _fused_kernel(x_ref, adj_ref, wgc_ref, bgc_ref, wlin_ref, blin_ref,
                  ne_ref, y_ref, support_ref):
    p = pl.program_id(0)
    i = pl.program_id(1)
    nb = pl.num_programs(1)

    @pl.when(jnp.logical_and(p == 0, i == 0))
    def _init_y():
        y_ref[...] = jnp.zeros_like(y_ref)

    @pl.when(p == 0)
    def _phase0():
        support_ref[pl.ds(i * BLK, BLK), :] = jnp.dot(
            x_ref[...], wgc_ref[...], preferred_element_type=jnp.float32)

    @pl.when(p == 1)
    def _phase1():
        gc = jnp.dot(adj_ref[...], support_ref[...],
                     preferred_element_type=jnp.float32)
        ne = jnp.maximum(gc + bgc_ref[...], 0.0)
        ne_ref[...] = ne
        m = jnp.max(ne, axis=1, keepdims=True)
        ls = ne - m - jnp.log(jnp.sum(jnp.exp(ne - m), axis=1, keepdims=True))
        y_ref[...] += jnp.sum(ls * wlin_ref[...], axis=0, keepdims=True)

    @pl.when(jnp.logical_and(p == 1, i == nb - 1))
    def _final_y():
        y_ref[...] += blin_ref[...]


@jax.jit
def kernel(x, adj, W_gc, b_gc, W_lin, b_lin):
    nb = N // BLK
    bgc2 = b_gc.reshape(1, NCLASS)
    wlin_t = W_lin.reshape(NFEAT, 1)
    blin2 = b_lin.reshape(1, 1)

    grid = (2, nb)
    ne, y = pl.pallas_call(
        _fused_kernel,
        grid=grid,
        in_specs=[
            pl.BlockSpec((BLK, NFEAT), lambda p, i: (jnp.where(p == 0, i, 0), 0)),
            pl.BlockSpec((BLK, N), lambda p, i: (jnp.where(p == 1, i, 0), 0)),
            pl.BlockSpec((NFEAT, NCLASS), lambda p, i: (0, 0)),
            pl.BlockSpec((1, NCLASS), lambda p, i: (0, 0)),
            pl.BlockSpec((BLK, 1), lambda p, i: (jnp.where(p == 1, i, 0), 0)),
            pl.BlockSpec((1, 1), lambda p, i: (0, 0)),
        ],
        out_specs=[
            pl.BlockSpec((BLK, NCLASS), lambda p, i: (jnp.where(p == 1, i, 0), 0)),
            pl.BlockSpec((1, NCLASS), lambda p, i: (0, 0)),
        ],
        out_shape=[
            jax.ShapeDtypeStruct((N, NCLASS), jnp.float32),
            jax.ShapeDtypeStruct((1, NCLASS), jnp.float32),
        ],
        scratch_shapes=[pltpu.VMEM((N, NCLASS), jnp.float32)],
    )(x, adj, W_gc, bgc2, wlin_t, blin2)
    return (y, ne)


# speedup vs baseline: 1.1031x; 1.1031x over previous
import functools

import jax
import jax.numpy as jnp
from jax.experimental import pallas as pl
from jax.experimental.pallas import tpu as pltpu

N = 4096
NFEAT = 4096
NCLASS = 8
BLK = 512


def _fused_kernel(x_ref, adj_ref, wgc_ref, bgc_ref, wlin_ref, blin_ref,
                  ne_ref, y_ref, support_ref):
    p = pl.program_id(0)
    i = pl.program_id(1)
    nb = pl.num_programs(1)

    @pl.when(jnp.logical_and(p == 0, i == 0))
    def _init_y():
        y_ref[...] = jnp.zeros_like(y_ref)

    @pl.when(p == 0)
    def _phase0():
        support_ref[pl.ds(i * BLK, BLK), :] = jnp.dot(
            x_ref[...], wgc_ref[...], preferred_element_type=jnp.float32)

    @pl.when(p == 1)
    def _phase1():
        gc = jnp.dot(adj_ref[...], support_ref[...],
                     preferred_element_type=jnp.float32)
        ne = jnp.maximum(gc + bgc_ref[...], 0.0)
        ne_ref[...] = ne
        m = jnp.max(ne, axis=1, keepdims=True)
        ls = ne - m - jnp.log(jnp.sum(jnp.exp(ne - m), axis=1, keepdims=True))
        y_ref[...] += jnp.sum(ls * wlin_ref[...], axis=0, keepdims=True)

    @pl.when(jnp.logical_and(p == 1, i == nb - 1))
    def _final_y():
        y_ref[...] += blin_ref[...]


@jax.jit
def kernel(x, adj, W_gc, b_gc, W_lin, b_lin):
    nb = N // BLK
    bgc2 = b_gc.reshape(1, NCLASS)
    wlin_t = W_lin.reshape(NFEAT, 1)
    blin2 = b_lin.reshape(1, 1)

    grid = (2, nb)
    ne, y = pl.pallas_call(
        _fused_kernel,
        grid=grid,
        in_specs=[
            pl.BlockSpec((BLK, NFEAT), lambda p, i: (jnp.where(p == 0, i, 0), 0)),
            pl.BlockSpec((BLK, N), lambda p, i: (jnp.where(p == 1, i, 0), 0)),
            pl.BlockSpec((NFEAT, NCLASS), lambda p, i: (0, 0)),
            pl.BlockSpec((1, NCLASS), lambda p, i: (0, 0)),
            pl.BlockSpec((BLK, 1), lambda p, i: (jnp.where(p == 1, i, 0), 0)),
            pl.BlockSpec((1, 1), lambda p, i: (0, 0)),
        ],
        out_specs=[
            pl.BlockSpec((BLK, NCLASS), lambda p, i: (jnp.where(p == 1, i, 0), 0)),
            pl.BlockSpec((1, NCLASS), lambda p, i: (0, 0)),
        ],
        out_shape=[
            jax.ShapeDtypeStruct((N, NCLASS), jnp.float32),
            jax.ShapeDtypeStruct((1, NCLASS), jnp.float32),
        ],
        scratch_shapes=[pltpu.VMEM((N, NCLASS), jnp.float32)],
    )(x, adj, W_gc, bgc2, wlin_t, blin2)
    return (y, ne)
